# baseline (speedup 1.0000x reference)
import functools

import jax
import jax.numpy as jnp
from jax import lax
from jax.experimental import pallas as pl
from jax.experimental.pallas import tpu as pltpu
from jax.experimental.pallas import tpu_sc as plsc

N, E, D, ED, H, G = 10000, 160000, 128, 16, 32, 64

NC, NS = 2, 16
NW = NC * NS
CH = 128
E_PAD = 163840
NPART = 2
EH = E_PAD // NPART
EPW = EH // NW
NCH = EPW // CH
IDXR = E_PAD // CH
N_PAD = 10240
NPS = N_PAD // NS

TILE_E = 2048
TILE_N = 1000
GRID_EH = EH // TILE_E
GRID_N = N // TILE_N

_PREC = lax.Precision.DEFAULT

_mesh = functools.partial(
    plsc.VectorSubcoreMesh,
    core_axis_name="c", subcore_axis_name="s", num_cores=NC, num_subcores=NS,
)

_SC_PARAMS = pltpu.CompilerParams(use_tc_tiling_on_sc=False)


def _zero_rows(buf, width):
    z = jnp.zeros((16,), jnp.float32)

    @pl.loop(0, CH)
    def _(r):
        for c0 in range(0, width, 16):
            buf[r, pl.ds(c0, 16)] = z


def _sc_gather(table, idx2, part, *, interpret=False):
    @functools.partial(
        pl.kernel,
        mesh=_mesh(),
        out_type=jax.ShapeDtypeStruct((EH, H), jnp.float32),
        scratch_types=[
            pltpu.VMEM((NCH, CH), jnp.int32),
            pltpu.VMEM((EPW, H), jnp.float32),
            pltpu.SemaphoreType.DMA,
            pltpu.SemaphoreType.DMA,
        ],
        compiler_params=_SC_PARAMS,
        interpret=interpret,
    )
    def k(table_hbm, idx_hbm, out_hbm, idx_v, rows_v, gsem, wsem):
        wid = lax.axis_index("s") * NC + lax.axis_index("c")
        base = wid * EPW
        pltpu.sync_copy(
            idx_hbm.at[pl.ds(part * (EH // CH) + wid * NCH, NCH)], idx_v)
        descs = []
        for jj in range(NCH):
            descs.append(pltpu.async_copy(
                table_hbm.at[idx_v.at[jj]],
                rows_v.at[pl.ds(jj * CH, CH)], gsem))
        for d in descs:
            d.wait()
        pltpu.async_copy(rows_v, out_hbm.at[pl.ds(base, EPW)], wsem).wait()

    return k(table, idx2)


def _sc_scatter(msg, dst2, part, *, interpret=False):
    out_types = (jax.ShapeDtypeStruct((NC, N_PAD, H), jnp.float32),
                 jax.ShapeDtypeStruct((NC, N_PAD, 16), jnp.float32))
    scratch_types = [
        pltpu.VMEM((NCH, CH), jnp.int32),
        pltpu.VMEM((EPW, H), jnp.float32),
        pltpu.VMEM_SHARED((N_PAD, H), jnp.float32),
        pltpu.SemaphoreType.DMA,
        pltpu.VMEM((CH, 16), jnp.float32),
        pltpu.VMEM_SHARED((N_PAD, 16), jnp.float32),
    ]

    @functools.partial(
        pl.kernel,
        mesh=_mesh(),
        out_type=out_types,
        scratch_types=scratch_types,
        compiler_params=_SC_PARAMS,
        interpret=interpret,
    )
    def k(msg_hbm, dst_hbm, out_hbm, degp_hbm, idx_v, msg_v, shared, sem,
          ones_v, shared_d):
        c = lax.axis_index("c")
        s = lax.axis_index("s")
        wid = s * NC + c
        base = wid * EPW

        _zero_rows(msg_v.at[pl.ds(0, CH)], H)

        @pl.loop(0, NPS // CH)
        def _(z):
            pltpu.sync_copy(msg_v.at[pl.ds(0, CH)],
                            shared.at[pl.ds(s * NPS + z * CH, CH)])

        _zero_rows(ones_v, 16)

        @pl.loop(0, NPS // CH)
        def _(z):
            pltpu.sync_copy(ones_v, shared_d.at[pl.ds(s * NPS + z * CH, CH)])

        one = jnp.ones((16,), jnp.float32)

        @pl.loop(0, CH)
        def _(r):
            ones_v[r, pl.ds(0, 16)] = one

        pltpu.sync_copy(
            dst_hbm.at[pl.ds(part * (EH // CH) + wid * NCH, NCH)], idx_v)
        pltpu.async_copy(msg_hbm.at[pl.ds(base, EPW)], msg_v, sem).wait()
        plsc.subcore_barrier()

        for jj in range(NCH):
            @pl.when(part * EH + base + jj * CH < E)
            def _():
                pltpu.sync_copy(msg_v.at[pl.ds(jj * CH, CH)],
                                shared.at[idx_v.at[jj]], add=True)
                pltpu.sync_copy(ones_v, shared_d.at[idx_v.at[jj]], add=True)

        plsc.subcore_barrier()
        pltpu.sync_copy(shared.at[pl.ds(s * NPS, NPS)],
                        out_hbm.at[c, pl.ds(s * NPS, NPS)])
        pltpu.sync_copy(shared_d.at[pl.ds(s * NPS, NPS)],
                        degp_hbm.at[c, pl.ds(s * NPS, NPS)])

    return k(msg, dst2)


def _tc_project(x, Wp, bp, *, interpret=False):
    def body(x_ref, wp_ref, bp_ref, out_ref):
        out_ref[...] = (
            jnp.dot(x_ref[...], wp_ref[...], precision=_PREC) + bp_ref[...])

    return pl.pallas_call(
        body,
        grid=(GRID_N,),
        in_specs=[
            pl.BlockSpec((TILE_N, D), lambda i: (i, 0)),
            pl.BlockSpec((D, H), lambda i: (0, 0)),
            pl.BlockSpec((1, H), lambda i: (0, 0)),
        ],
        out_specs=pl.BlockSpec((TILE_N, H), lambda i: (i, 0)),
        out_shape=jax.ShapeDtypeStruct((N, H), jnp.float32),
        interpret=interpret,
    )(x, Wp, bp.reshape(1, H))


def _tc_msg(ea4, hs4, W1l, b1l, W2mT, B2T, *, interpret=False):
    T4 = TILE_E // 4

    def body(ea_ref, hs_ref, w1t_ref, b1_ref, w2mt_ref, b2t_ref, out_ref):
        eaT = ea_ref[...].T
        hsT = hs_ref[...].T
        ea_t = jnp.concatenate(
            [eaT[j * ED:(j + 1) * ED, :] for j in range(4)], axis=1)
        hs_t = jnp.concatenate(
            [hsT[j * H:(j + 1) * H, :] for j in range(4)], axis=1)
        z = jnp.dot(w1t_ref[...], ea_t, precision=_PREC) + b1_ref[...]
        t_t = z * jax.nn.sigmoid(z)
        u_t = jnp.concatenate(
            [(t_t[k:k + 1, :] * hs_t).astype(jnp.bfloat16)
             for k in range(H)], axis=0)
        msg_t = (jnp.dot(w2mt_ref[...], u_t,
                         preferred_element_type=jnp.float32)
                 + jnp.dot(b2t_ref[...], hs_t, precision=_PREC))
        m4 = jnp.concatenate(
            [msg_t[:, j * T4:(j + 1) * T4] for j in range(4)], axis=0)
        out_ref[...] = m4.T

    return pl.pallas_call(
        body,
        grid=(GRID_EH,),
        in_specs=[
            pl.BlockSpec((T4, 4 * ED), lambda i: (i, 0)),
            pl.BlockSpec((T4, 4 * H), lambda i: (i, 0)),
            pl.BlockSpec((H, ED), lambda i: (0, 0)),
            pl.BlockSpec((H, 1), lambda i: (0, 0)),
            pl.BlockSpec((H, H * H), lambda i: (0, 0)),
            pl.BlockSpec((H, H), lambda i: (0, 0)),
        ],
        out_specs=pl.BlockSpec((T4, 4 * H), lambda i: (i, 0)),
        out_shape=jax.ShapeDtypeStruct((EH // 4, 4 * H), jnp.float32),
        interpret=interpret,
    )(ea4, hs4, W1l.T, b1l.reshape(H, 1), W2mT, B2T)


def _deg_col(da_ref, db_ref):
    d = (da_ref[0, :, 0:1] + da_ref[1, :, 0:1]
         + db_ref[0, :, 0:1] + db_ref[1, :, 0:1])
    return jnp.clip(d, 1.0, None)


def _tc_update(pa, pb, da, db, h, rootl, cbiasl, *, interpret=False):
    def body(pa_ref, pb_ref, da_ref, db_ref, h_ref, root_ref, cb_ref,
             out_ref):
        aggr = pa_ref[0] + pa_ref[1] + pb_ref[0] + pb_ref[1]
        z = (aggr / _deg_col(da_ref, db_ref)
             + jnp.dot(h_ref[...], root_ref[...], precision=_PREC)
             + cb_ref[...])
        out_ref[...] = z * jax.nn.sigmoid(z)

    return pl.pallas_call(
        body,
        grid=(GRID_N,),
        in_specs=[
            pl.BlockSpec((NC, TILE_N, H), lambda i: (0, i, 0)),
            pl.BlockSpec((NC, TILE_N, H), lambda i: (0, i, 0)),
            pl.BlockSpec((NC, TILE_N, 16), lambda i: (0, i, 0)),
            pl.BlockSpec((NC, TILE_N, 16), lambda i: (0, i, 0)),
            pl.BlockSpec((TILE_N, H), lambda i: (i, 0)),
            pl.BlockSpec((H, H), lambda i: (0, 0)),
            pl.BlockSpec((1, H), lambda i: (0, 0)),
        ],
        out_specs=pl.BlockSpec((TILE_N, H), lambda i: (i, 0)),
        out_shape=jax.ShapeDtypeStruct((N, H), jnp.float32),
        interpret=interpret,
    )(pa, pb, da, db, h, rootl, cbiasl.reshape(1, H))


def _tc_update_pool(pa, pb, da, db, h, rootl, cbiasl, batch3, Wh1, bh1,
                    Wh2, bh2, *, interpret=False):
    def body(pa_ref, pb_ref, da_ref, db_ref, h_ref, root_ref, cb_ref, b_ref,
             wh1_ref, bh1_ref, wh2_ref, bh2_ref, out_ref, hg_acc, cnt_acc):
        i = pl.program_id(0)

        @pl.when(i == 0)
        def _():
            hg_acc[...] = jnp.zeros_like(hg_acc)
            cnt_acc[...] = jnp.zeros_like(cnt_acc)

        aggr = pa_ref[0] + pa_ref[1] + pb_ref[0] + pb_ref[1]
        z = (aggr / _deg_col(da_ref, db_ref)
             + jnp.dot(h_ref[...], root_ref[...], precision=_PREC)
             + cb_ref[...])
        hn = z * jax.nn.sigmoid(z)

        b = b_ref[0, 0, :]
        onehot = (b[:, None]
                  == lax.broadcasted_iota(jnp.int32, (1, G), 1)
                  ).astype(jnp.float32)
        hg_acc[...] += lax.dot_general(
            onehot, hn, (((0,), (0,)), ((), ())), precision=_PREC)
        cnt_acc[...] += lax.dot_general(
            onehot, jnp.ones((TILE_N, 1), jnp.float32),
            (((0,), (0,)), ((), ())), precision=_PREC)

        @pl.when(i == GRID_N - 1)
        def _():
            hg = hg_acc[...] / jnp.clip(cnt_acc[...], 1.0, None)
            g1 = jnp.dot(hg, wh1_ref[...], precision=_PREC) + bh1_ref[...]
            g1 = g1 * jax.nn.sigmoid(g1)
            out_ref[...] = (
                jnp.dot(g1, wh2_ref[...], precision=_PREC) + bh2_ref[...])

    return pl.pallas_call(
        body,
        grid=(GRID_N,),
        in_specs=[
            pl.BlockSpec((NC, TILE_N, H), lambda i: (0, i, 0)),
            pl.BlockSpec((NC, TILE_N, H), lambda i: (0, i, 0)),
            pl.BlockSpec((NC, TILE_N, 16), lambda i: (0, i, 0)),
            pl.BlockSpec((NC, TILE_N, 16), lambda i: (0, i, 0)),
            pl.BlockSpec((TILE_N, H), lambda i: (i, 0)),
            pl.BlockSpec((H, H), lambda i: (0, 0)),
            pl.BlockSpec((1, H), lambda i: (0, 0)),
            pl.BlockSpec((1, 1, TILE_N), lambda i: (i, 0, 0)),
            pl.BlockSpec((H, H), lambda i: (0, 0)),
            pl.BlockSpec((1, H), lambda i: (0, 0)),
            pl.BlockSpec((H, 1), lambda i: (0, 0)),
            pl.BlockSpec((1, 1), lambda i: (0, 0)),
        ],
        out_specs=pl.BlockSpec((G, 1), lambda i: (0, 0)),
        out_shape=jax.ShapeDtypeStruct((G, 1), jnp.float32),
        scratch_shapes=[
            pltpu.VMEM((G, H), jnp.float32),
            pltpu.VMEM((G, 1), jnp.float32),
        ],
        interpret=interpret,
    )(pa, pb, da, db, h, rootl, cbiasl.reshape(1, H), batch3,
      Wh1, bh1.reshape(1, H), Wh2, bh2.reshape(1, 1))


def kernel(x, edge_attr, Wp, bp, W1, b1, W2, b2, root, cbias, Wh1, bh1,
           Wh2, bh2, edge_index, batch):
    src = edge_index[0]
    dst = edge_index[1]

    pad_e = E_PAD - E
    ea4s = [
        edge_attr[:EH].reshape(EH // 4, 4 * ED),
        jnp.concatenate(
            [edge_attr[EH:].reshape((E - EH) // 4, 4 * ED),
             jnp.zeros((pad_e // 4, 4 * ED), edge_attr.dtype)], axis=0),
    ]
    src2 = jnp.concatenate(
        [src, jnp.zeros((pad_e,), src.dtype)]).reshape(IDXR, CH)
    dst2 = jnp.concatenate(
        [dst, jnp.zeros((pad_e,), dst.dtype)]).reshape(IDXR, CH)
    batch3 = batch.reshape(GRID_N, 1, TILE_N)

    L = W1.shape[0]
    W2mT = [W2[l].reshape(H, H, H).reshape(H * H, H).T.astype(jnp.bfloat16)
            for l in range(L)]
    B2T = [b2[l].reshape(H, H).T for l in range(L)]

    h = _tc_project(x, Wp, bp)
    for l in range(L):
        parts = []
        for p in range(NPART):
            hs = _sc_gather(h, src2, p)
            msg4 = _tc_msg(ea4s[p], hs.reshape(EH // 4, 4 * H),
                           W1[l], b1[l], W2mT[l], B2T[l])
            parts.append(_sc_scatter(msg4.reshape(EH, H), dst2, p))
        (pa, da), (pb, db) = parts
        if l < L - 1:
            h = _tc_update(pa, pb, da, db, h, root[l], cbias[l])
        else:
            out = _tc_update_pool(pa, pb, da, db, h, root[l], cbias[l],
                                  batch3, Wh1, bh1, Wh2, bh2)
    return out.reshape(-1)


# speedup vs baseline: 1.0253x; 1.0253x over previous
import functools

import jax
import jax.numpy as jnp
from jax import lax
from jax.experimental import pallas as pl
from jax.experimental.pallas import tpu as pltpu
from jax.experimental.pallas import tpu_sc as plsc

N, E, D, ED, H, G = 10000, 160000, 128, 16, 32, 64

NC, NS = 2, 16
NW = NC * NS
CH = 128
E_PAD = 163840
NPART = 2
EH = E_PAD // NPART
EPW = EH // NW
NCH = EPW // CH
IDXR = E_PAD // CH
N_PAD = 10240
NPS = N_PAD // NS

TILE_E = 4096
TILE_N = 1000
GRID_EH = EH // TILE_E
GRID_N = N // TILE_N

_PREC = lax.Precision.DEFAULT

_mesh = functools.partial(
    plsc.VectorSubcoreMesh,
    core_axis_name="c", subcore_axis_name="s", num_cores=NC, num_subcores=NS,
)

_SC_PARAMS = pltpu.CompilerParams(use_tc_tiling_on_sc=False)


def _zero_rows(buf, width):
    z = jnp.zeros((16,), jnp.float32)

    @pl.loop(0, CH)
    def _(r):
        for c0 in range(0, width, 16):
            buf[r, pl.ds(c0, 16)] = z


def _sc_gather(table, idx2, part, *, interpret=False):
    @functools.partial(
        pl.kernel,
        mesh=_mesh(),
        out_type=jax.ShapeDtypeStruct((EH, H), jnp.float32),
        scratch_types=[
            pltpu.VMEM((NCH, CH), jnp.int32),
            pltpu.VMEM((EPW, H), jnp.float32),
            pltpu.SemaphoreType.DMA,
            pltpu.SemaphoreType.DMA,
        ],
        compiler_params=_SC_PARAMS,
        interpret=interpret,
    )
    def k(table_hbm, idx_hbm, out_hbm, idx_v, rows_v, gsem, wsem):
        wid = lax.axis_index("s") * NC + lax.axis_index("c")
        base = wid * EPW
        pltpu.sync_copy(
            idx_hbm.at[pl.ds(part * (EH // CH) + wid * NCH, NCH)], idx_v)
        descs = []
        for jj in range(NCH):
            descs.append(pltpu.async_copy(
                table_hbm.at[idx_v.at[jj]],
                rows_v.at[pl.ds(jj * CH, CH)], gsem))
        for d in descs:
            d.wait()
        pltpu.async_copy(rows_v, out_hbm.at[pl.ds(base, EPW)], wsem).wait()

    return k(table, idx2)


def _sc_scatter(msg, dst2, part, *, interpret=False):
    out_types = (jax.ShapeDtypeStruct((NC, N_PAD, H), jnp.float32),
                 jax.ShapeDtypeStruct((NC, N_PAD, 16), jnp.float32))
    scratch_types = [
        pltpu.VMEM((NCH, CH), jnp.int32),
        pltpu.VMEM((EPW, H), jnp.float32),
        pltpu.VMEM_SHARED((N_PAD, H), jnp.float32),
        pltpu.SemaphoreType.DMA,
        pltpu.VMEM((CH, 16), jnp.float32),
        pltpu.VMEM_SHARED((N_PAD, 16), jnp.float32),
    ]

    @functools.partial(
        pl.kernel,
        mesh=_mesh(),
        out_type=out_types,
        scratch_types=scratch_types,
        compiler_params=_SC_PARAMS,
        interpret=interpret,
    )
    def k(msg_hbm, dst_hbm, out_hbm, degp_hbm, idx_v, msg_v, shared, sem,
          ones_v, shared_d):
        c = lax.axis_index("c")
        s = lax.axis_index("s")
        wid = s * NC + c
        base = wid * EPW

        _zero_rows(msg_v.at[pl.ds(0, CH)], H)

        @pl.loop(0, NPS // CH)
        def _(z):
            pltpu.sync_copy(msg_v.at[pl.ds(0, CH)],
                            shared.at[pl.ds(s * NPS + z * CH, CH)])

        _zero_rows(ones_v, 16)

        @pl.loop(0, NPS // CH)
        def _(z):
            pltpu.sync_copy(ones_v, shared_d.at[pl.ds(s * NPS + z * CH, CH)])

        one = jnp.ones((16,), jnp.float32)

        @pl.loop(0, CH)
        def _(r):
            ones_v[r, pl.ds(0, 16)] = one

        pltpu.sync_copy(
            dst_hbm.at[pl.ds(part * (EH // CH) + wid * NCH, NCH)], idx_v)
        pltpu.async_copy(msg_hbm.at[pl.ds(base, EPW)], msg_v, sem).wait()
        plsc.subcore_barrier()

        for jj in range(NCH):
            @pl.when(part * EH + base + jj * CH < E)
            def _():
                pltpu.sync_copy(msg_v.at[pl.ds(jj * CH, CH)],
                                shared.at[idx_v.at[jj]], add=True)
                pltpu.sync_copy(ones_v, shared_d.at[idx_v.at[jj]], add=True)

        plsc.subcore_barrier()
        pltpu.sync_copy(shared.at[pl.ds(s * NPS, NPS)],
                        out_hbm.at[c, pl.ds(s * NPS, NPS)])
        pltpu.sync_copy(shared_d.at[pl.ds(s * NPS, NPS)],
                        degp_hbm.at[c, pl.ds(s * NPS, NPS)])

    return k(msg, dst2)


def _tc_project(x, Wp, bp, *, interpret=False):
    def body(x_ref, wp_ref, bp_ref, out_ref):
        out_ref[...] = (
            jnp.dot(x_ref[...], wp_ref[...], precision=_PREC) + bp_ref[...])

    return pl.pallas_call(
        body,
        grid=(GRID_N,),
        in_specs=[
            pl.BlockSpec((TILE_N, D), lambda i: (i, 0)),
            pl.BlockSpec((D, H), lambda i: (0, 0)),
            pl.BlockSpec((1, H), lambda i: (0, 0)),
        ],
        out_specs=pl.BlockSpec((TILE_N, H), lambda i: (i, 0)),
        out_shape=jax.ShapeDtypeStruct((N, H), jnp.float32),
        interpret=interpret,
    )(x, Wp, bp.reshape(1, H))


def _tc_msg(ea4, hs4, W1l, b1l, W2mT, B2T, *, interpret=False):
    T4 = TILE_E // 4

    def body(ea_ref, hs_ref, w1t_ref, b1_ref, w2mt_ref, b2t_ref, out_ref):
        eaT = ea_ref[...].T
        hsT = hs_ref[...].T
        ea_t = jnp.concatenate(
            [eaT[j * ED:(j + 1) * ED, :] for j in range(4)], axis=1)
        hs_t = jnp.concatenate(
            [hsT[j * H:(j + 1) * H, :] for j in range(4)], axis=1)
        z = jnp.dot(w1t_ref[...], ea_t, precision=_PREC) + b1_ref[...]
        t_t = z * jax.nn.sigmoid(z)
        u_t = jnp.concatenate(
            [(t_t[k:k + 1, :] * hs_t).astype(jnp.bfloat16)
             for k in range(H)], axis=0)
        msg_t = (jnp.dot(w2mt_ref[...], u_t,
                         preferred_element_type=jnp.float32)
                 + jnp.dot(b2t_ref[...], hs_t, precision=_PREC))
        m4 = jnp.concatenate(
            [msg_t[:, j * T4:(j + 1) * T4] for j in range(4)], axis=0)
        out_ref[...] = m4.T

    return pl.pallas_call(
        body,
        grid=(GRID_EH,),
        in_specs=[
            pl.BlockSpec((T4, 4 * ED), lambda i: (i, 0)),
            pl.BlockSpec((T4, 4 * H), lambda i: (i, 0)),
            pl.BlockSpec((H, ED), lambda i: (0, 0)),
            pl.BlockSpec((H, 1), lambda i: (0, 0)),
            pl.BlockSpec((H, H * H), lambda i: (0, 0)),
            pl.BlockSpec((H, H), lambda i: (0, 0)),
        ],
        out_specs=pl.BlockSpec((T4, 4 * H), lambda i: (i, 0)),
        out_shape=jax.ShapeDtypeStruct((EH // 4, 4 * H), jnp.float32),
        interpret=interpret,
    )(ea4, hs4, W1l.T, b1l.reshape(H, 1), W2mT, B2T)


def _deg_col(da_ref, db_ref):
    d = (da_ref[0, :, 0:1] + da_ref[1, :, 0:1]
         + db_ref[0, :, 0:1] + db_ref[1, :, 0:1])
    return jnp.clip(d, 1.0, None)


def _tc_update(pa, pb, da, db, h, rootl, cbiasl, *, interpret=False):
    def body(pa_ref, pb_ref, da_ref, db_ref, h_ref, root_ref, cb_ref,
             out_ref):
        aggr = pa_ref[0] + pa_ref[1] + pb_ref[0] + pb_ref[1]
        z = (aggr / _deg_col(da_ref, db_ref)
             + jnp.dot(h_ref[...], root_ref[...], precision=_PREC)
             + cb_ref[...])
        out_ref[...] = z * jax.nn.sigmoid(z)

    return pl.pallas_call(
        body,
        grid=(GRID_N,),
        in_specs=[
            pl.BlockSpec((NC, TILE_N, H), lambda i: (0, i, 0)),
            pl.BlockSpec((NC, TILE_N, H), lambda i: (0, i, 0)),
            pl.BlockSpec((NC, TILE_N, 16), lambda i: (0, i, 0)),
            pl.BlockSpec((NC, TILE_N, 16), lambda i: (0, i, 0)),
            pl.BlockSpec((TILE_N, H), lambda i: (i, 0)),
            pl.BlockSpec((H, H), lambda i: (0, 0)),
            pl.BlockSpec((1, H), lambda i: (0, 0)),
        ],
        out_specs=pl.BlockSpec((TILE_N, H), lambda i: (i, 0)),
        out_shape=jax.ShapeDtypeStruct((N, H), jnp.float32),
        interpret=interpret,
    )(pa, pb, da, db, h, rootl, cbiasl.reshape(1, H))


def _tc_update_pool(pa, pb, da, db, h, rootl, cbiasl, batch3, Wh1, bh1,
                    Wh2, bh2, *, interpret=False):
    def body(pa_ref, pb_ref, da_ref, db_ref, h_ref, root_ref, cb_ref, b_ref,
             wh1_ref, bh1_ref, wh2_ref, bh2_ref, out_ref, hg_acc, cnt_acc):
        i = pl.program_id(0)

        @pl.when(i == 0)
        def _():
            hg_acc[...] = jnp.zeros_like(hg_acc)
            cnt_acc[...] = jnp.zeros_like(cnt_acc)

        aggr = pa_ref[0] + pa_ref[1] + pb_ref[0] + pb_ref[1]
        z = (aggr / _deg_col(da_ref, db_ref)
             + jnp.dot(h_ref[...], root_ref[...], precision=_PREC)
             + cb_ref[...])
        hn = z * jax.nn.sigmoid(z)

        b = b_ref[0, 0, :]
        onehot = (b[:, None]
                  == lax.broadcasted_iota(jnp.int32, (1, G), 1)
                  ).astype(jnp.float32)
        hg_acc[...] += lax.dot_general(
            onehot, hn, (((0,), (0,)), ((), ())), precision=_PREC)
        cnt_acc[...] += lax.dot_general(
            onehot, jnp.ones((TILE_N, 1), jnp.float32),
            (((0,), (0,)), ((), ())), precision=_PREC)

        @pl.when(i == GRID_N - 1)
        def _():
            hg = hg_acc[...] / jnp.clip(cnt_acc[...], 1.0, None)
            g1 = jnp.dot(hg, wh1_ref[...], precision=_PREC) + bh1_ref[...]
            g1 = g1 * jax.nn.sigmoid(g1)
            out_ref[...] = (
                jnp.dot(g1, wh2_ref[...], precision=_PREC) + bh2_ref[...])

    return pl.pallas_call(
        body,
        grid=(GRID_N,),
        in_specs=[
            pl.BlockSpec((NC, TILE_N, H), lambda i: (0, i, 0)),
            pl.BlockSpec((NC, TILE_N, H), lambda i: (0, i, 0)),
            pl.BlockSpec((NC, TILE_N, 16), lambda i: (0, i, 0)),
            pl.BlockSpec((NC, TILE_N, 16), lambda i: (0, i, 0)),
            pl.BlockSpec((TILE_N, H), lambda i: (i, 0)),
            pl.BlockSpec((H, H), lambda i: (0, 0)),
            pl.BlockSpec((1, H), lambda i: (0, 0)),
            pl.BlockSpec((1, 1, TILE_N), lambda i: (i, 0, 0)),
            pl.BlockSpec((H, H), lambda i: (0, 0)),
            pl.BlockSpec((1, H), lambda i: (0, 0)),
            pl.BlockSpec((H, 1), lambda i: (0, 0)),
            pl.BlockSpec((1, 1), lambda i: (0, 0)),
        ],
        out_specs=pl.BlockSpec((G, 1), lambda i: (0, 0)),
        out_shape=jax.ShapeDtypeStruct((G, 1), jnp.float32),
        scratch_shapes=[
            pltpu.VMEM((G, H), jnp.float32),
            pltpu.VMEM((G, 1), jnp.float32),
        ],
        interpret=interpret,
    )(pa, pb, da, db, h, rootl, cbiasl.reshape(1, H), batch3,
      Wh1, bh1.reshape(1, H), Wh2, bh2.reshape(1, 1))


def kernel(x, edge_attr, Wp, bp, W1, b1, W2, b2, root, cbias, Wh1, bh1,
           Wh2, bh2, edge_index, batch):
    src = edge_index[0]
    dst = edge_index[1]

    pad_e = E_PAD - E
    ea4 = jnp.concatenate(
        [edge_attr.reshape(E // 4, 4 * ED),
         jnp.zeros((pad_e // 4, 4 * ED), edge_attr.dtype)], axis=0)
    ea4s = [ea4[:EH // 4], ea4[EH // 4:]]
    src2 = jnp.concatenate(
        [src, jnp.zeros((pad_e,), src.dtype)]).reshape(IDXR, CH)
    dst2 = jnp.concatenate(
        [dst, jnp.zeros((pad_e,), dst.dtype)]).reshape(IDXR, CH)
    batch3 = batch.reshape(GRID_N, 1, TILE_N)

    L = W1.shape[0]
    W2mT = [W2[l].reshape(H, H, H).reshape(H * H, H).T.astype(jnp.bfloat16)
            for l in range(L)]
    B2T = [b2[l].reshape(H, H).T for l in range(L)]

    h = _tc_project(x, Wp, bp)
    for l in range(L):
        parts = []
        for p in range(NPART):
            hs = _sc_gather(h, src2, p)
            msg4 = _tc_msg(ea4s[p], hs.reshape(EH // 4, 4 * H),
                           W1[l], b1[l], W2mT[l], B2T[l])
            parts.append(_sc_scatter(msg4.reshape(EH, H), dst2, p))
        (pa, da), (pb, db) = parts
        if l < L - 1:
            h = _tc_update(pa, pb, da, db, h, root[l], cbias[l])
        else:
            out = _tc_update_pool(pa, pb, da, db, h, root[l], cbias[l],
                                  batch3, Wh1, bh1, Wh2, bh2)
    return out.reshape(-1)


# speedup vs baseline: 1.0668x; 1.0405x over previous
import functools

import jax
import jax.numpy as jnp
from jax import lax
from jax.experimental import pallas as pl
from jax.experimental.pallas import tpu as pltpu
from jax.experimental.pallas import tpu_sc as plsc

N, E, D, ED, H, G = 10000, 160000, 128, 16, 32, 64

NC, NS = 2, 16
NW = NC * NS
CH = 128
E_PAD = 163840
NPART = 2
EH = E_PAD // NPART
EPW = EH // NW
NCH = EPW // CH
IDXR = E_PAD // CH
N_PAD = 10240
NPS = N_PAD // NS

TILE_E = 2048
TILE_N = 1000
GRID_EH = EH // TILE_E
GRID_N = N // TILE_N

_PREC = lax.Precision.DEFAULT

_mesh = functools.partial(
    plsc.VectorSubcoreMesh,
    core_axis_name="c", subcore_axis_name="s", num_cores=NC, num_subcores=NS,
)

_SC_PARAMS = pltpu.CompilerParams(use_tc_tiling_on_sc=False)


def _zero_rows(buf, width):
    z = jnp.zeros((16,), jnp.float32)

    @pl.loop(0, CH)
    def _(r):
        for c0 in range(0, width, 16):
            buf[r, pl.ds(c0, 16)] = z


def _sc_gather(table, idx2, part, *, interpret=False):
    @functools.partial(
        pl.kernel,
        mesh=_mesh(),
        out_type=jax.ShapeDtypeStruct((EH, H), jnp.float32),
        scratch_types=[
            pltpu.VMEM((NCH, CH), jnp.int32),
            pltpu.VMEM((EPW, H), jnp.float32),
            pltpu.SemaphoreType.DMA,
            pltpu.SemaphoreType.DMA,
        ],
        compiler_params=_SC_PARAMS,
        interpret=interpret,
    )
    def k(table_hbm, idx_hbm, out_hbm, idx_v, rows_v, gsem, wsem):
        wid = lax.axis_index("s") * NC + lax.axis_index("c")
        base = wid * EPW
        pltpu.sync_copy(
            idx_hbm.at[pl.ds(part * (EH // CH) + wid * NCH, NCH)], idx_v)
        descs = []
        for jj in range(NCH):
            descs.append(pltpu.async_copy(
                table_hbm.at[idx_v.at[jj]],
                rows_v.at[pl.ds(jj * CH, CH)], gsem))
        for d in descs:
            d.wait()
        pltpu.async_copy(rows_v, out_hbm.at[pl.ds(base, EPW)], wsem).wait()

    return k(table, idx2)


def _sc_scatter(msg, dst2, part, *, interpret=False):
    out_types = (jax.ShapeDtypeStruct((NC, N_PAD, H), jnp.float32),
                 jax.ShapeDtypeStruct((NC, N_PAD, 16), jnp.float32))
    scratch_types = [
        pltpu.VMEM((NCH, CH), jnp.int32),
        pltpu.VMEM((EPW, H), jnp.float32),
        pltpu.VMEM_SHARED((N_PAD, H), jnp.float32),
        pltpu.SemaphoreType.DMA,
        pltpu.VMEM((CH, 16), jnp.float32),
        pltpu.VMEM_SHARED((N_PAD, 16), jnp.float32),
    ]

    @functools.partial(
        pl.kernel,
        mesh=_mesh(),
        out_type=out_types,
        scratch_types=scratch_types,
        compiler_params=_SC_PARAMS,
        interpret=interpret,
    )
    def k(msg_hbm, dst_hbm, out_hbm, degp_hbm, idx_v, msg_v, shared, sem,
          ones_v, shared_d):
        c = lax.axis_index("c")
        s = lax.axis_index("s")
        wid = s * NC + c
        base = wid * EPW

        _zero_rows(msg_v.at[pl.ds(0, CH)], H)

        @pl.loop(0, NPS // CH)
        def _(z):
            pltpu.sync_copy(msg_v.at[pl.ds(0, CH)],
                            shared.at[pl.ds(s * NPS + z * CH, CH)])

        _zero_rows(ones_v, 16)

        @pl.loop(0, NPS // CH)
        def _(z):
            pltpu.sync_copy(ones_v, shared_d.at[pl.ds(s * NPS + z * CH, CH)])

        one = jnp.ones((16,), jnp.float32)

        @pl.loop(0, CH)
        def _(r):
            ones_v[r, pl.ds(0, 16)] = one

        pltpu.sync_copy(
            dst_hbm.at[pl.ds(part * (EH // CH) + wid * NCH, NCH)], idx_v)
        pltpu.async_copy(msg_hbm.at[pl.ds(base, EPW)], msg_v, sem).wait()
        plsc.subcore_barrier()

        for jj in range(NCH):
            @pl.when(part * EH + base + jj * CH < E)
            def _():
                pltpu.sync_copy(msg_v.at[pl.ds(jj * CH, CH)],
                                shared.at[idx_v.at[jj]], add=True)
                pltpu.sync_copy(ones_v, shared_d.at[idx_v.at[jj]], add=True)

        plsc.subcore_barrier()
        pltpu.sync_copy(shared.at[pl.ds(s * NPS, NPS)],
                        out_hbm.at[c, pl.ds(s * NPS, NPS)])
        pltpu.sync_copy(shared_d.at[pl.ds(s * NPS, NPS)],
                        degp_hbm.at[c, pl.ds(s * NPS, NPS)])

    return k(msg, dst2)


def _tc_project(x, Wp, bp, *, interpret=False):
    def body(x_ref, wp_ref, bp_ref, out_ref):
        out_ref[...] = (
            jnp.dot(x_ref[...], wp_ref[...], precision=_PREC) + bp_ref[...])

    return pl.pallas_call(
        body,
        grid=(GRID_N,),
        in_specs=[
            pl.BlockSpec((TILE_N, D), lambda i: (i, 0)),
            pl.BlockSpec((D, H), lambda i: (0, 0)),
            pl.BlockSpec((1, H), lambda i: (0, 0)),
        ],
        out_specs=pl.BlockSpec((TILE_N, H), lambda i: (i, 0)),
        out_shape=jax.ShapeDtypeStruct((N, H), jnp.float32),
        interpret=interpret,
    )(x, Wp, bp.reshape(1, H))


def _tc_msg(ea4, hs4, W1l, b1l, W2mT, B2T, part, *, interpret=False):
    T4 = TILE_E // 4

    def body(ea_ref, hs_ref, w1t_ref, b1_ref, w2mt_ref, b2t_ref, out_ref):
        eaT = ea_ref[...].T
        hsT = hs_ref[...].T
        ea_t = jnp.concatenate(
            [eaT[j * ED:(j + 1) * ED, :] for j in range(4)], axis=1)
        hs_t = jnp.concatenate(
            [hsT[j * H:(j + 1) * H, :] for j in range(4)], axis=1)
        z = jnp.dot(w1t_ref[...], ea_t, precision=_PREC) + b1_ref[...]
        t_t = z * jax.nn.sigmoid(z)
        u_t = jnp.concatenate(
            [(t_t[k:k + 1, :] * hs_t).astype(jnp.bfloat16)
             for k in range(H)], axis=0)
        msg_t = (jnp.dot(w2mt_ref[...], u_t,
                         preferred_element_type=jnp.float32)
                 + jnp.dot(b2t_ref[...], hs_t, precision=_PREC))
        m4 = jnp.concatenate(
            [msg_t[:, j * T4:(j + 1) * T4] for j in range(4)], axis=0)
        out_ref[...] = m4.T

    poff = part * GRID_EH
    return pl.pallas_call(
        body,
        grid=(GRID_EH,),
        in_specs=[
            pl.BlockSpec((T4, 4 * ED), lambda i: (i + poff, 0)),
            pl.BlockSpec((T4, 4 * H), lambda i: (i, 0)),
            pl.BlockSpec((H, ED), lambda i: (0, 0)),
            pl.BlockSpec((H, 1), lambda i: (0, 0)),
            pl.BlockSpec((H, H * H), lambda i: (0, 0)),
            pl.BlockSpec((H, H), lambda i: (0, 0)),
        ],
        out_specs=pl.BlockSpec((T4, 4 * H), lambda i: (i, 0)),
        out_shape=jax.ShapeDtypeStruct((EH // 4, 4 * H), jnp.float32),
        interpret=interpret,
    )(ea4, hs4, W1l.T, b1l.reshape(H, 1), W2mT, B2T)


def _deg_col(da_ref, db_ref):
    d = (da_ref[0, :, 0:1] + da_ref[1, :, 0:1]
         + db_ref[0, :, 0:1] + db_ref[1, :, 0:1])
    return jnp.clip(d, 1.0, None)


def _tc_update(pa, pb, da, db, h, rootl, cbiasl, *, interpret=False):
    def body(pa_ref, pb_ref, da_ref, db_ref, h_ref, root_ref, cb_ref,
             out_ref):
        aggr = pa_ref[0] + pa_ref[1] + pb_ref[0] + pb_ref[1]
        z = (aggr / _deg_col(da_ref, db_ref)
             + jnp.dot(h_ref[...], root_ref[...], precision=_PREC)
             + cb_ref[...])
        out_ref[...] = z * jax.nn.sigmoid(z)

    return pl.pallas_call(
        body,
        grid=(GRID_N,),
        in_specs=[
            pl.BlockSpec((NC, TILE_N, H), lambda i: (0, i, 0)),
            pl.BlockSpec((NC, TILE_N, H), lambda i: (0, i, 0)),
            pl.BlockSpec((NC, TILE_N, 16), lambda i: (0, i, 0)),
            pl.BlockSpec((NC, TILE_N, 16), lambda i: (0, i, 0)),
            pl.BlockSpec((TILE_N, H), lambda i: (i, 0)),
            pl.BlockSpec((H, H), lambda i: (0, 0)),
            pl.BlockSpec((1, H), lambda i: (0, 0)),
        ],
        out_specs=pl.BlockSpec((TILE_N, H), lambda i: (i, 0)),
        out_shape=jax.ShapeDtypeStruct((N, H), jnp.float32),
        interpret=interpret,
    )(pa, pb, da, db, h, rootl, cbiasl.reshape(1, H))


def _tc_update_pool(pa, pb, da, db, h, rootl, cbiasl, batch3, Wh1, bh1,
                    Wh2, bh2, *, interpret=False):
    def body(pa_ref, pb_ref, da_ref, db_ref, h_ref, root_ref, cb_ref, b_ref,
             wh1_ref, bh1_ref, wh2_ref, bh2_ref, out_ref, hg_acc, cnt_acc):
        i = pl.program_id(0)

        @pl.when(i == 0)
        def _():
            hg_acc[...] = jnp.zeros_like(hg_acc)
            cnt_acc[...] = jnp.zeros_like(cnt_acc)

        aggr = pa_ref[0] + pa_ref[1] + pb_ref[0] + pb_ref[1]
        z = (aggr / _deg_col(da_ref, db_ref)
             + jnp.dot(h_ref[...], root_ref[...], precision=_PREC)
             + cb_ref[...])
        hn = z * jax.nn.sigmoid(z)

        b = b_ref[0, 0, :]
        onehot = (b[:, None]
                  == lax.broadcasted_iota(jnp.int32, (1, G), 1)
                  ).astype(jnp.float32)
        hg_acc[...] += lax.dot_general(
            onehot, hn, (((0,), (0,)), ((), ())), precision=_PREC)
        cnt_acc[...] += lax.dot_general(
            onehot, jnp.ones((TILE_N, 1), jnp.float32),
            (((0,), (0,)), ((), ())), precision=_PREC)

        @pl.when(i == GRID_N - 1)
        def _():
            hg = hg_acc[...] / jnp.clip(cnt_acc[...], 1.0, None)
            g1 = jnp.dot(hg, wh1_ref[...], precision=_PREC) + bh1_ref[...]
            g1 = g1 * jax.nn.sigmoid(g1)
            out_ref[...] = (
                jnp.dot(g1, wh2_ref[...], precision=_PREC) + bh2_ref[...])

    return pl.pallas_call(
        body,
        grid=(GRID_N,),
        in_specs=[
            pl.BlockSpec((NC, TILE_N, H), lambda i: (0, i, 0)),
            pl.BlockSpec((NC, TILE_N, H), lambda i: (0, i, 0)),
            pl.BlockSpec((NC, TILE_N, 16), lambda i: (0, i, 0)),
            pl.BlockSpec((NC, TILE_N, 16), lambda i: (0, i, 0)),
            pl.BlockSpec((TILE_N, H), lambda i: (i, 0)),
            pl.BlockSpec((H, H), lambda i: (0, 0)),
            pl.BlockSpec((1, H), lambda i: (0, 0)),
            pl.BlockSpec((1, 1, TILE_N), lambda i: (i, 0, 0)),
            pl.BlockSpec((H, H), lambda i: (0, 0)),
            pl.BlockSpec((1, H), lambda i: (0, 0)),
            pl.BlockSpec((H, 1), lambda i: (0, 0)),
            pl.BlockSpec((1, 1), lambda i: (0, 0)),
        ],
        out_specs=pl.BlockSpec((G, 1), lambda i: (0, 0)),
        out_shape=jax.ShapeDtypeStruct((G, 1), jnp.float32),
        scratch_shapes=[
            pltpu.VMEM((G, H), jnp.float32),
            pltpu.VMEM((G, 1), jnp.float32),
        ],
        interpret=interpret,
    )(pa, pb, da, db, h, rootl, cbiasl.reshape(1, H), batch3,
      Wh1, bh1.reshape(1, H), Wh2, bh2.reshape(1, 1))


def kernel(x, edge_attr, Wp, bp, W1, b1, W2, b2, root, cbias, Wh1, bh1,
           Wh2, bh2, edge_index, batch):
    src = edge_index[0]
    dst = edge_index[1]

    pad_e = E_PAD - E
    ea4 = jnp.concatenate(
        [edge_attr.reshape(E // 4, 4 * ED),
         jnp.zeros((pad_e // 4, 4 * ED), edge_attr.dtype)], axis=0)
    src2 = jnp.concatenate(
        [src, jnp.zeros((pad_e,), src.dtype)]).reshape(IDXR, CH)
    dst2 = jnp.concatenate(
        [dst, jnp.zeros((pad_e,), dst.dtype)]).reshape(IDXR, CH)
    batch3 = batch.reshape(GRID_N, 1, TILE_N)

    L = W1.shape[0]
    W2mT = [W2[l].reshape(H, H, H).reshape(H * H, H).T.astype(jnp.bfloat16)
            for l in range(L)]
    B2T = [b2[l].reshape(H, H).T for l in range(L)]

    h = _tc_project(x, Wp, bp)
    for l in range(L):
        parts = []
        for p in range(NPART):
            hs = _sc_gather(h, src2, p)
            msg4 = _tc_msg(ea4, hs.reshape(EH // 4, 4 * H),
                           W1[l], b1[l], W2mT[l], B2T[l], p)
            parts.append(_sc_scatter(msg4.reshape(EH, H), dst2, p))
        (pa, da), (pb, db) = parts
        if l < L - 1:
            h = _tc_update(pa, pb, da, db, h, root[l], cbias[l])
        else:
            out = _tc_update_pool(pa, pb, da, db, h, root[l], cbias[l],
                                  batch3, Wh1, bh1, Wh2, bh2)
    return out.reshape(-1)
